# baseline (speedup 1.0000x reference)
import functools

import jax
import jax.numpy as jnp
from jax.experimental import pallas as pl
from jax.experimental.pallas import tpu as pltpu

_N = 4096
_D = 16
_CBLK = 512


_BIG = 2**30


def _cost_block_kernel(x_ref, c_ref, m_ref, a_ref):
    j = pl.program_id(0)
    x = x_ref[...]
    c = c_ref[...]
    xsq = jnp.sum(x * x, axis=1)
    csq = jnp.sum(c * c, axis=1)
    g = jax.lax.dot_general(
        x, c, (((1,), (1,)), ((), ())), preferred_element_type=jnp.float32
    )
    d2 = (xsq[:, None] + csq[None, :]) - 2.0 * g
    d2 = jnp.maximum(d2, 0.0)
    mb = jnp.min(d2, axis=1)
    iota = jax.lax.broadcasted_iota(jnp.int32, d2.shape, 1)
    ab = jnp.min(jnp.where(d2 == mb[:, None], iota, _BIG), axis=1) + j * _CBLK

    @pl.when(j == 0)
    def _():
        m_ref[...] = mb
        a_ref[...] = ab

    @pl.when(j > 0)
    def _():
        m = m_ref[...]
        better = mb < m
        m_ref[...] = jnp.where(better, mb, m)
        a_ref[...] = jnp.where(better, ab, a_ref[...])


@functools.partial(jax.jit, static_argnames=("ncols",))
def _cost(x, centers, ncols):
    grid = (ncols // _CBLK,)
    return pl.pallas_call(
        _cost_block_kernel,
        grid=grid,
        in_specs=[
            pl.BlockSpec((_N, _D), lambda j: (0, 0)),
            pl.BlockSpec((_CBLK, _D), lambda j: (j, 0)),
        ],
        out_specs=[
            pl.BlockSpec((_N,), lambda j: (0,)),
            pl.BlockSpec((_N,), lambda j: (0,)),
        ],
        out_shape=[
            jax.ShapeDtypeStruct((_N,), jnp.float32),
            jax.ShapeDtypeStruct((_N,), jnp.int32),
        ],
    )(x, centers)


def kernel(x, centers):
    costs, idx = _cost(x, centers, 512)
    s0 = jnp.sum(costs)
    max_iters = 200
    centers1 = jnp.take(x, idx, axis=0)
    costs1, idx1 = _cost(x, centers1, _N)
    s1 = jnp.sum(costs1)
    run_min = jnp.minimum(s0, s1)

    def cond(carry):
        _, _, _, _, it, stop = carry
        return jnp.logical_and(jnp.logical_not(stop), it < max_iters)

    def body(carry):
        idx_c, run_min_c, best_c, best_i, it, _ = carry
        ncenters = jnp.take(x, idx_c, axis=0)
        ncosts, nidx = _cost(x, ncenters, _N)
        s = jnp.sum(ncosts)
        stop = s == run_min_c
        better = s < run_min_c
        best_c = jnp.where(better, ncenters, best_c)
        best_i = jnp.where(better, nidx, best_i)
        run_min_c = jnp.minimum(run_min_c, s)
        return (nidx, run_min_c, best_c, best_i, it + 1, stop)

    init = (idx1, run_min, centers1, idx1, jnp.int32(1), jnp.bool_(False))
    _, _, best_c, best_i, _, _ = jax.lax.while_loop(cond, body, init)
    return (best_c, best_i)


# speedup vs baseline: 1.9508x; 1.9508x over previous
import functools

import jax
import jax.numpy as jnp
from jax.experimental import pallas as pl

_N = 4096
_D = 16
_CBLK = 512
_BIG = 2**30


def _cost_block_kernel(x_ref, c_ref, m_ref, a_ref, s_ref):
    j = pl.program_id(0)
    nblk = pl.num_programs(0)
    x = x_ref[...]
    c = c_ref[...]
    xsq = jnp.sum(x * x, axis=1)
    csq = jnp.sum(c * c, axis=1, keepdims=True)
    g = jax.lax.dot_general(
        c, x, (((1,), (1,)), ((), ())), preferred_element_type=jnp.float32
    )
    d2 = (csq + xsq[None, :]) - 2.0 * g
    mb = jnp.min(d2, axis=0)
    iota = jax.lax.broadcasted_iota(jnp.int32, d2.shape, 0) + j * _CBLK
    ab = jnp.min(jnp.where(d2 == mb[None, :], iota, _BIG), axis=0)

    @pl.when(j == 0)
    def _():
        m_ref[...] = mb
        a_ref[...] = ab

    @pl.when(j > 0)
    def _():
        m = m_ref[...]
        better = mb < m
        m_ref[...] = jnp.where(better, mb, m)
        a_ref[...] = jnp.where(better, ab, a_ref[...])

    @pl.when(j == nblk - 1)
    def _():
        s_ref[...] = jnp.sum(jnp.maximum(m_ref[...], 0.0)).reshape(1, 1)


@functools.partial(jax.jit, static_argnames=("ncols",))
def _cost(x, centers, ncols):
    grid = (ncols // _CBLK,)
    _, a, s = pl.pallas_call(
        _cost_block_kernel,
        grid=grid,
        in_specs=[
            pl.BlockSpec((_N, _D), lambda j: (0, 0)),
            pl.BlockSpec((_CBLK, _D), lambda j: (j, 0)),
        ],
        out_specs=[
            pl.BlockSpec((_N,), lambda j: (0,)),
            pl.BlockSpec((_N,), lambda j: (0,)),
            pl.BlockSpec((1, 1), lambda j: (0, 0)),
        ],
        out_shape=[
            jax.ShapeDtypeStruct((_N,), jnp.float32),
            jax.ShapeDtypeStruct((_N,), jnp.int32),
            jax.ShapeDtypeStruct((1, 1), jnp.float32),
        ],
    )(x, centers)
    return s[0, 0], a


def kernel(x, centers):
    s0, idx = _cost(x, centers, 512)
    max_iters = 200
    centers1 = jnp.take(x, idx, axis=0)
    s1, idx1 = _cost(x, centers1, _N)
    run_min = jnp.minimum(s0, s1)

    def cond(carry):
        _, _, _, _, it, stop = carry
        return jnp.logical_and(jnp.logical_not(stop), it < max_iters)

    def body(carry):
        idx_c, run_min_c, best_p, best_i, it, _ = carry
        ncenters = jnp.take(x, idx_c, axis=0)
        s, nidx = _cost(x, ncenters, _N)
        stop = s == run_min_c
        better = s < run_min_c
        best_p = jnp.where(better, idx_c, best_p)
        best_i = jnp.where(better, nidx, best_i)
        run_min_c = jnp.minimum(run_min_c, s)
        return (nidx, run_min_c, best_p, best_i, it + 1, stop)

    init = (idx1, run_min, idx, idx1, jnp.int32(1), jnp.bool_(False))
    _, _, best_p, best_i, _, _ = jax.lax.while_loop(cond, body, init)
    best_c = jnp.take(x, best_p, axis=0)
    return (best_c, best_i)


# speedup vs baseline: 2.4050x; 1.2329x over previous
import functools

import jax
import jax.numpy as jnp
from jax.experimental import pallas as pl

_N = 4096
_D = 16
_CBLK = 512
_BIG = 2**30


def _cost_block_kernel(x_ref, c_ref, m_ref, a_ref, s_ref):
    j = pl.program_id(0)
    nblk = pl.num_programs(0)
    x = x_ref[...]
    c = c_ref[...]
    xsq = jnp.sum(x * x, axis=1)
    csq = jnp.sum(c * c, axis=1, keepdims=True)
    g2 = jax.lax.dot_general(
        c * (-2.0), x, (((1,), (1,)), ((), ())), preferred_element_type=jnp.float32
    )
    d2 = (csq + xsq[None, :]) + g2

    d2r = d2.reshape(_CBLK // 8, 8, _N)
    m8 = jnp.full((8, _N), jnp.inf, jnp.float32)
    a8 = jnp.zeros((8, _N), jnp.int32)
    for gi in range(_CBLK // 8):
        v = d2r[gi]
        lt = v < m8
        a8 = jnp.where(lt, gi, a8)
        m8 = jnp.where(lt, v, m8)
    mb = jnp.min(m8, axis=0)
    subiota = jax.lax.broadcasted_iota(jnp.int32, (8, _N), 0)
    jfull = a8 * 8 + subiota + j * _CBLK
    ab = jnp.min(jnp.where(m8 == mb[None, :], jfull, _BIG), axis=0)

    @pl.when(j == 0)
    def _():
        m_ref[...] = mb
        a_ref[...] = ab

    @pl.when(j > 0)
    def _():
        m = m_ref[...]
        better = mb < m
        m_ref[...] = jnp.where(better, mb, m)
        a_ref[...] = jnp.where(better, ab, a_ref[...])

    @pl.when(j == nblk - 1)
    def _():
        s_ref[...] = jnp.sum(jnp.maximum(m_ref[...], 0.0)).reshape(1, 1)


@functools.partial(jax.jit, static_argnames=("ncols",))
def _cost(x, centers, ncols):
    grid = (ncols // _CBLK,)
    _, a, s = pl.pallas_call(
        _cost_block_kernel,
        grid=grid,
        in_specs=[
            pl.BlockSpec((_N, _D), lambda j: (0, 0)),
            pl.BlockSpec((_CBLK, _D), lambda j: (j, 0)),
        ],
        out_specs=[
            pl.BlockSpec((_N,), lambda j: (0,)),
            pl.BlockSpec((_N,), lambda j: (0,)),
            pl.BlockSpec((1, 1), lambda j: (0, 0)),
        ],
        out_shape=[
            jax.ShapeDtypeStruct((_N,), jnp.float32),
            jax.ShapeDtypeStruct((_N,), jnp.int32),
            jax.ShapeDtypeStruct((1, 1), jnp.float32),
        ],
    )(x, centers)
    return s[0, 0], a


def kernel(x, centers):
    s0, idx = _cost(x, centers, 512)
    max_iters = 200
    centers1 = jnp.take(x, idx, axis=0)
    s1, idx1 = _cost(x, centers1, _N)
    run_min = jnp.minimum(s0, s1)

    def cond(carry):
        _, _, _, _, it, stop = carry
        return jnp.logical_and(jnp.logical_not(stop), it < max_iters)

    def body(carry):
        idx_c, run_min_c, best_p, best_i, it, _ = carry
        ncenters = jnp.take(x, idx_c, axis=0)
        s, nidx = _cost(x, ncenters, _N)
        stop = s == run_min_c
        better = s < run_min_c
        best_p = jnp.where(better, idx_c, best_p)
        best_i = jnp.where(better, nidx, best_i)
        run_min_c = jnp.minimum(run_min_c, s)
        return (nidx, run_min_c, best_p, best_i, it + 1, stop)

    init = (idx1, run_min, idx, idx1, jnp.int32(1), jnp.bool_(False))
    _, _, best_p, best_i, _, _ = jax.lax.while_loop(cond, body, init)
    best_c = jnp.take(x, best_p, axis=0)
    return (best_c, best_i)


# speedup vs baseline: 2.6510x; 1.1023x over previous
import functools

import jax
import jax.numpy as jnp
from jax.experimental import pallas as pl

_N = 4096
_D = 16
_CBLK = 1024
_BIG = 2**30


def _cost_block_kernel(x_ref, c_ref, m_ref, a_ref, s_ref):
    j = pl.program_id(0)
    nblk = pl.num_programs(0)
    x = x_ref[...]
    c = c_ref[...]
    xsq = jnp.sum(x * x, axis=1)
    csq = jnp.sum(c * c, axis=1, keepdims=True)
    g2 = jax.lax.dot_general(
        c * (-2.0), x, (((1,), (1,)), ((), ())), preferred_element_type=jnp.float32
    )
    d2 = (csq + xsq[None, :]) + g2

    cb = c.shape[0]
    d2r = d2.reshape(cb // 8, 8, _N)
    m8 = jnp.full((8, _N), jnp.inf, jnp.float32)
    a8 = jnp.zeros((8, _N), jnp.int32)
    for gi in range(cb // 8):
        v = d2r[gi]
        lt = v < m8
        a8 = jnp.where(lt, gi, a8)
        m8 = jnp.where(lt, v, m8)
    mb = jnp.min(m8, axis=0)
    subiota = jax.lax.broadcasted_iota(jnp.int32, (8, _N), 0)
    jfull = a8 * 8 + subiota + j * cb
    ab = jnp.min(jnp.where(m8 == mb[None, :], jfull, _BIG), axis=0)

    @pl.when(j == 0)
    def _():
        m_ref[...] = mb
        a_ref[...] = ab

    @pl.when(j > 0)
    def _():
        m = m_ref[...]
        better = mb < m
        m_ref[...] = jnp.where(better, mb, m)
        a_ref[...] = jnp.where(better, ab, a_ref[...])

    @pl.when(j == nblk - 1)
    def _():
        s_ref[...] = jnp.sum(jnp.maximum(m_ref[...], 0.0)).reshape(1, 1)


@functools.partial(jax.jit, static_argnames=("ncols",))
def _cost(x, centers, ncols):
    blk = min(_CBLK, ncols)
    grid = (ncols // blk,)
    _, a, s = pl.pallas_call(
        _cost_block_kernel,
        grid=grid,
        in_specs=[
            pl.BlockSpec((_N, _D), lambda j: (0, 0)),
            pl.BlockSpec((blk, _D), lambda j: (j, 0)),
        ],
        out_specs=[
            pl.BlockSpec((_N,), lambda j: (0,)),
            pl.BlockSpec((_N,), lambda j: (0,)),
            pl.BlockSpec((1, 1), lambda j: (0, 0)),
        ],
        out_shape=[
            jax.ShapeDtypeStruct((_N,), jnp.float32),
            jax.ShapeDtypeStruct((_N,), jnp.int32),
            jax.ShapeDtypeStruct((1, 1), jnp.float32),
        ],
    )(x, centers)
    return s[0, 0], a


def kernel(x, centers):
    s0, idx = _cost(x, centers, 512)
    max_iters = 200
    centers1 = jnp.take(x, idx, axis=0)
    s1, idx1 = _cost(x, centers1, _N)
    run_min = jnp.minimum(s0, s1)

    def cond(carry):
        _, _, _, _, it, stop = carry
        return jnp.logical_and(jnp.logical_not(stop), it < max_iters)

    def body(carry):
        idx_c, run_min_c, best_p, best_i, it, _ = carry
        ncenters = jnp.take(x, idx_c, axis=0)
        s, nidx = _cost(x, ncenters, _N)
        stop = s == run_min_c
        better = s < run_min_c
        best_p = jnp.where(better, idx_c, best_p)
        best_i = jnp.where(better, nidx, best_i)
        run_min_c = jnp.minimum(run_min_c, s)
        return (nidx, run_min_c, best_p, best_i, it + 1, stop)

    init = (idx1, run_min, idx, idx1, jnp.int32(1), jnp.bool_(False))
    _, _, best_p, best_i, _, _ = jax.lax.while_loop(cond, body, init)
    best_c = jnp.take(x, best_p, axis=0)
    return (best_c, best_i)


# speedup vs baseline: 2.7742x; 1.0465x over previous
import functools

import jax
import jax.numpy as jnp
from jax.experimental import pallas as pl

_N = 4096
_D = 16
_CBLK = 2048
_BIG = 2**30


def _cost_block_kernel(x_ref, c_ref, m_ref, a_ref, s_ref):
    j = pl.program_id(0)
    nblk = pl.num_programs(0)
    x = x_ref[...]
    c = c_ref[...]
    xsq = jnp.sum(x * x, axis=1)
    csq = jnp.sum(c * c, axis=1, keepdims=True)
    g2 = jax.lax.dot_general(
        c * (-2.0), x, (((1,), (1,)), ((), ())), preferred_element_type=jnp.float32
    )
    d2 = (csq + xsq[None, :]) + g2

    cb = c.shape[0]
    d2r = d2.reshape(cb // 8, 8, _N)
    m8 = jnp.full((8, _N), jnp.inf, jnp.float32)
    a8 = jnp.zeros((8, _N), jnp.int32)
    for gi in range(cb // 8):
        v = d2r[gi]
        lt = v < m8
        a8 = jnp.where(lt, gi, a8)
        m8 = jnp.where(lt, v, m8)
    mb = jnp.min(m8, axis=0)
    subiota = jax.lax.broadcasted_iota(jnp.int32, (8, _N), 0)
    jfull = a8 * 8 + subiota + j * cb
    ab = jnp.min(jnp.where(m8 == mb[None, :], jfull, _BIG), axis=0)

    @pl.when(j == 0)
    def _():
        m_ref[...] = mb
        a_ref[...] = ab

    @pl.when(j > 0)
    def _():
        m = m_ref[...]
        better = mb < m
        m_ref[...] = jnp.where(better, mb, m)
        a_ref[...] = jnp.where(better, ab, a_ref[...])

    @pl.when(j == nblk - 1)
    def _():
        s_ref[...] = jnp.sum(jnp.maximum(m_ref[...], 0.0)).reshape(1, 1)


@functools.partial(jax.jit, static_argnames=("ncols",))
def _cost(x, centers, ncols):
    blk = min(_CBLK, ncols)
    grid = (ncols // blk,)
    _, a, s = pl.pallas_call(
        _cost_block_kernel,
        grid=grid,
        in_specs=[
            pl.BlockSpec((_N, _D), lambda j: (0, 0)),
            pl.BlockSpec((blk, _D), lambda j: (j, 0)),
        ],
        out_specs=[
            pl.BlockSpec((_N,), lambda j: (0,)),
            pl.BlockSpec((_N,), lambda j: (0,)),
            pl.BlockSpec((1, 1), lambda j: (0, 0)),
        ],
        out_shape=[
            jax.ShapeDtypeStruct((_N,), jnp.float32),
            jax.ShapeDtypeStruct((_N,), jnp.int32),
            jax.ShapeDtypeStruct((1, 1), jnp.float32),
        ],
    )(x, centers)
    return s[0, 0], a


def kernel(x, centers):
    s0, idx = _cost(x, centers, 512)
    max_iters = 200
    centers1 = jnp.take(x, idx, axis=0)
    s1, idx1 = _cost(x, centers1, _N)
    run_min = jnp.minimum(s0, s1)

    def cond(carry):
        _, _, _, _, it, stop = carry
        return jnp.logical_and(jnp.logical_not(stop), it < max_iters)

    def body(carry):
        idx_c, run_min_c, best_p, best_i, it, _ = carry
        ncenters = jnp.take(x, idx_c, axis=0)
        s, nidx = _cost(x, ncenters, _N)
        stop = s == run_min_c
        better = s < run_min_c
        best_p = jnp.where(better, idx_c, best_p)
        best_i = jnp.where(better, nidx, best_i)
        run_min_c = jnp.minimum(run_min_c, s)
        return (nidx, run_min_c, best_p, best_i, it + 1, stop)

    init = (idx1, run_min, idx, idx1, jnp.int32(1), jnp.bool_(False))
    _, _, best_p, best_i, _, _ = jax.lax.while_loop(cond, body, init)
    best_c = jnp.take(x, best_p, axis=0)
    return (best_c, best_i)


# speedup vs baseline: 2.8916x; 1.0423x over previous
import functools

import jax
import jax.numpy as jnp
from jax.experimental import pallas as pl

_N = 4096
_D = 16
_CBLK = 2048
_BIG = 2**30


def _cost_block_kernel(x_ref, c_ref, m_ref, a_ref, s_ref):
    j = pl.program_id(0)
    nblk = pl.num_programs(0)
    x = x_ref[...]
    c = c_ref[...]
    xsq = jnp.sum(x * x, axis=1)
    csq = jnp.sum(c * c, axis=1, keepdims=True)
    g2 = jax.lax.dot_general(
        c * (-2.0), x, (((1,), (1,)), ((), ())), preferred_element_type=jnp.float32
    )
    d2 = (csq + xsq[None, :]) + g2

    cb = c.shape[0]
    d2r = d2.reshape(cb // 8, 8, _N)
    m8 = jnp.full((8, _N), jnp.inf, jnp.float32)
    a8 = jnp.zeros((8, _N), jnp.int32)
    for gi in range(cb // 8):
        v = d2r[gi]
        lt = v < m8
        a8 = jnp.where(lt, gi, a8)
        m8 = jnp.where(lt, v, m8)
    mb = jnp.min(m8, axis=0)
    subiota = jax.lax.broadcasted_iota(jnp.int32, (8, _N), 0)
    jfull = a8 * 8 + subiota + j * cb
    ab = jnp.min(jnp.where(m8 == mb[None, :], jfull, _BIG), axis=0)

    @pl.when(j == 0)
    def _():
        m_ref[...] = mb
        a_ref[...] = ab

    @pl.when(j > 0)
    def _():
        m = m_ref[...]
        better = mb < m
        m_ref[...] = jnp.where(better, mb, m)
        a_ref[...] = jnp.where(better, ab, a_ref[...])

    @pl.when(j == nblk - 1)
    def _():
        s_ref[...] = jnp.sum(jnp.maximum(m_ref[...], 0.0)).reshape(1, 1)


@functools.partial(jax.jit, static_argnames=("ncols",))
def _cost(x, centers, ncols):
    blk = min(_CBLK, ncols)
    grid = (ncols // blk,)
    _, a, s = pl.pallas_call(
        _cost_block_kernel,
        grid=grid,
        in_specs=[
            pl.BlockSpec((_N, _D), lambda j: (0, 0)),
            pl.BlockSpec((blk, _D), lambda j: (j, 0)),
        ],
        out_specs=[
            pl.BlockSpec((_N,), lambda j: (0,)),
            pl.BlockSpec((_N,), lambda j: (0,)),
            pl.BlockSpec((1, 1), lambda j: (0, 0)),
        ],
        out_shape=[
            jax.ShapeDtypeStruct((_N,), jnp.float32),
            jax.ShapeDtypeStruct((_N,), jnp.int32),
            jax.ShapeDtypeStruct((1, 1), jnp.float32),
        ],
    )(x, centers)
    return s[0, 0], a


def _step_block_kernel(x_ref, c_ref, idx_ref, rm_ref, bp_ref, bi_ref,
                       m_ref, a_ref, rmo_ref, stop_ref, bpo_ref, bio_ref):
    _cost_block_kernel(x_ref, c_ref, m_ref, a_ref, rmo_ref)
    j = pl.program_id(0)
    nblk = pl.num_programs(0)

    @pl.when(j == nblk - 1)
    def _():
        s = rmo_ref[0, 0]
        rm = rm_ref[0, 0]
        better = s < rm
        stop_ref[...] = jnp.where(s == rm, 1, 0).reshape(1, 1)
        rmo_ref[...] = jnp.minimum(rm, s).reshape(1, 1)
        bpo_ref[...] = jnp.where(better, idx_ref[...], bp_ref[...])
        bio_ref[...] = jnp.where(better, a_ref[...], bi_ref[...])


@jax.jit
def _step(x, centers, idx_c, run_min, best_p, best_i):
    grid = (_N // _CBLK,)
    _, a, rm, stop, bp, bi = pl.pallas_call(
        _step_block_kernel,
        grid=grid,
        in_specs=[
            pl.BlockSpec((_N, _D), lambda j: (0, 0)),
            pl.BlockSpec((_CBLK, _D), lambda j: (j, 0)),
            pl.BlockSpec((_N,), lambda j: (0,)),
            pl.BlockSpec((1, 1), lambda j: (0, 0)),
            pl.BlockSpec((_N,), lambda j: (0,)),
            pl.BlockSpec((_N,), lambda j: (0,)),
        ],
        out_specs=[
            pl.BlockSpec((_N,), lambda j: (0,)),
            pl.BlockSpec((_N,), lambda j: (0,)),
            pl.BlockSpec((1, 1), lambda j: (0, 0)),
            pl.BlockSpec((1, 1), lambda j: (0, 0)),
            pl.BlockSpec((_N,), lambda j: (0,)),
            pl.BlockSpec((_N,), lambda j: (0,)),
        ],
        out_shape=[
            jax.ShapeDtypeStruct((_N,), jnp.float32),
            jax.ShapeDtypeStruct((_N,), jnp.int32),
            jax.ShapeDtypeStruct((1, 1), jnp.float32),
            jax.ShapeDtypeStruct((1, 1), jnp.int32),
            jax.ShapeDtypeStruct((_N,), jnp.int32),
            jax.ShapeDtypeStruct((_N,), jnp.int32),
        ],
    )(x, centers, idx_c, run_min, best_p, best_i)
    return a, rm, stop[0, 0] == 1, bp, bi


def kernel(x, centers):
    s0, idx = _cost(x, centers, 512)
    max_iters = 200
    centers1 = jnp.take(x, idx, axis=0)
    s1, idx1 = _cost(x, centers1, _N)
    run_min = jnp.minimum(s0, s1)

    def cond(carry):
        _, _, _, _, it, stop = carry
        return jnp.logical_and(jnp.logical_not(stop), it < max_iters)

    def body(carry):
        idx_c, run_min_c, best_p, best_i, it, _ = carry
        ncenters = jnp.take(x, idx_c, axis=0)
        nidx, rm, stop, best_p, best_i = _step(
            x, ncenters, idx_c, run_min_c.reshape(1, 1), best_p, best_i
        )
        return (nidx, rm[0, 0], best_p, best_i, it + 1, stop)

    init = (idx1, run_min, idx, idx1, jnp.int32(1), jnp.bool_(False))
    _, _, best_p, best_i, _, _ = jax.lax.while_loop(cond, body, init)
    best_c = jnp.take(x, best_p, axis=0)
    return (best_c, best_i)
